# baseline (speedup 1.0000x reference)
import functools

import jax
import jax.numpy as jnp
from jax import lax
from jax.experimental import pallas as pl
from jax.experimental.pallas import tpu as pltpu
from jax.experimental.pallas import tpu_sc as plsc

N_NODES = 16384
N_EDGES = 262144
D = 256
N_GRAPHS = 64
N_SC = 2
N_TILES = 16
NW = N_SC * N_TILES
SEG = 4096
NSEG = N_EDGES // SEG
BR = 256
NPASS = N_NODES // (BR * NW)
GR = 16
NBUF = 4
ACCW = (BR + 1) * D
DR = 512
ROWBLK = 1024

_mesh = plsc.VectorSubcoreMesh(core_axis_name="c", subcore_axis_name="s")
_sc_params = pltpu.CompilerParams(needs_layout_passes=False)


@functools.partial(
    pl.kernel,
    out_type=jax.ShapeDtypeStruct((N_NODES, 16), jnp.float32),
    mesh=_mesh,
    scratch_types=[
        pltpu.VMEM((2, SEG), jnp.int32),
        pltpu.VMEM((16 * DR + 16,), jnp.float32),
        pltpu.VMEM((DR, 16), jnp.float32),
        pltpu.SemaphoreType.DMA,
        pltpu.SemaphoreType.DMA,
    ],
    compiler_params=_sc_params,
)
def _sc_degree(dst_hbm, deg_hbm, dstb, hist, degrep, sem0, sem1):
    c = lax.axis_index("c")
    s = lax.axis_index("s")
    wid = c * N_TILES + s
    base = wid * DR

    def zero_hist(i, _):
        hist[pl.ds(i * 16, 16)] = jnp.zeros((16,), jnp.float32)
        return 0

    lax.fori_loop(0, (16 * DR + 16) // 16, zero_hist, 0)

    def start_seg(seg, k, sem):
        pltpu.async_copy(dst_hbm.at[pl.ds(seg * SEG, SEG)], dstb.at[k], sem)

    def wait_seg(seg, k, sem):
        pltpu.make_async_copy(dst_hbm.at[pl.ds(seg * SEG, SEG)],
                              dstb.at[k], sem).wait()

    ones = jnp.ones((16,), jnp.float32)
    lanes = lax.iota(jnp.int32, 16)

    def scan(k):
        def body(v, _):
            dv = dstb[k, pl.ds(v * 16, 16)]
            m = (dv >= base) & (dv < base + DR)
            idx = jnp.where(m, lanes * DR + (dv - base), 16 * DR + lanes)
            plsc.addupdate_scatter(hist, [idx], ones)
            return 0

        lax.fori_loop(0, SEG // 16, body, 0)

    start_seg(0, 0, sem0)

    def seg_body(i, _):
        g0 = i * 2
        start_seg(g0 + 1, 1, sem1)
        wait_seg(g0, 0, sem0)
        scan(0)

        @pl.when(g0 + 2 < NSEG)
        def _():
            start_seg(g0 + 2, 0, sem0)

        wait_seg(g0 + 1, 1, sem1)
        scan(1)
        return 0

    lax.fori_loop(0, NSEG // 2, seg_body, 0)

    for j in range(DR // 16):
        acc = hist[pl.ds(j * 16, 16)]
        for l in range(1, 16):
            acc = acc + hist[pl.ds(l * DR + j * 16, 16)]
        for l in range(16):
            degrep[j * 16 + l, pl.ds(0, 16)] = jnp.full((16,), acc[l],
                                                        jnp.float32)
    pltpu.sync_copy(degrep, deg_hbm.at[pl.ds(base, DR)])


@functools.partial(
    pl.kernel,
    out_type=jax.ShapeDtypeStruct((N_NODES * D,), jnp.float32),
    mesh=_mesh,
    scratch_types=[
        pltpu.VMEM((2, SEG), jnp.int32),
        pltpu.VMEM((2, SEG), jnp.int32),
        pltpu.VMEM((SEG + 128,), jnp.int32),
        pltpu.VMEM((SEG + 128,), jnp.int32),
        pltpu.VMEM((SEG + 128,), jnp.int32),
        pltpu.VMEM((SEG + 128,), jnp.int32),
    ] + [pltpu.VMEM((GR, D), jnp.float32)] * NBUF + [
        pltpu.VMEM((ACCW,), jnp.float32),
    ] + [pltpu.SemaphoreType.DMA] * (2 + NBUF),
    compiler_params=_sc_params,
)
def _sc_propagate(y_hbm, src_hbm, dst_hbm, zacc_hbm, out_hbm,
                  srcb, dstb, sidx0, dloc0, sidx1, dloc1, *rest):
    banks = ((sidx0, dloc0), (sidx1, dloc1))
    rbufs = rest[:NBUF]
    acc = rest[NBUF]
    sem_s0, sem_s1 = rest[NBUF + 1], rest[NBUF + 2]
    gsems = rest[NBUF + 3:]
    c = lax.axis_index("c")
    s = lax.axis_index("s")
    wid = c * N_TILES + s
    lanes = lax.iota(jnp.int32, 16)

    def start_seg(seg, k, sem):
        pltpu.async_copy(src_hbm.at[pl.ds(seg * SEG, SEG)], srcb.at[k], sem)
        pltpu.async_copy(dst_hbm.at[pl.ds(seg * SEG, SEG)], dstb.at[k], sem)

    def wait_seg(seg, k, sem):
        pltpu.make_async_copy(src_hbm.at[pl.ds(seg * SEG, SEG)],
                              srcb.at[k], sem).wait()
        pltpu.make_async_copy(dst_hbm.at[pl.ds(seg * SEG, SEG)],
                              dstb.at[k], sem).wait()

    def start_g(sidx, g, rbuf, sem):
        pltpu.async_copy(y_hbm.at[sidx.at[pl.ds(g * GR, GR)]], rbuf, sem)

    def wait_g(sidx, g, rbuf, sem):
        pltpu.make_async_copy(y_hbm.at[sidx.at[pl.ds(g * GR, GR)]],
                              rbuf, sem).wait()

    def accumulate(dloc, rbuf, g):
        def rbody(r, _):
            dl16 = dloc[pl.ds(g * GR + r, 16)]
            off = dl16[0] * D
            for j in range(D // 16):
                vec = rbuf[r, pl.ds(j * 16, 16)]
                plsc.addupdate(acc.at[pl.ds(off + j * 16, 16)], vec)
            return 0

        lax.fori_loop(0, GR, rbody, 0)

    def scan_into(b, k, base):
        sidx, dloc = banks[b]

        def scan_body(v, cnt):
            dv = dstb[k, pl.ds(v * 16, 16)]
            m = (dv >= base) & (dv < base + BR)

            def slow(c):
                sv = srcb[k, pl.ds(v * 16, 16)]
                mi = m.astype(jnp.int32)
                incl = plsc.cumsum(mi)
                pos = jnp.where(m, c + incl - mi, SEG + lanes)
                plsc.store_scatter(sidx, [pos], sv)
                plsc.store_scatter(dloc, [pos], dv - base)
                return c + incl[15]

            return lax.cond(jnp.any(m), slow, lambda c: c, cnt)

        cnt = lax.fori_loop(0, SEG // 16, scan_body, 0)
        for pvi in range(GR // 16):
            sidx[pl.ds(cnt + pvi * 16, 16)] = jnp.zeros((16,), jnp.int32)
            dloc[pl.ds(cnt + pvi * 16, 16)] = jnp.full((16,), BR, jnp.int32)
        return (cnt + GR - 1) // GR

    def fire_first(b, ng):
        sidx, _ = banks[b]
        for kk in range(NBUF - 1):
            @pl.when(kk < ng)
            def _(kk=kk):
                start_g(sidx, kk, rbufs[kk], gsems[kk])

    def drain(b, ng):
        sidx, dloc = banks[b]

        def inner(jj, _):
            for kk in range(NBUF):
                g = jj * NBUF + kk

                @pl.when(g + NBUF - 1 < ng)
                def _(g=g, kk=kk):
                    start_g(sidx, g + NBUF - 1, rbufs[(kk + NBUF - 1) % NBUF],
                            gsems[(kk + NBUF - 1) % NBUF])

                @pl.when(g < ng)
                def _(g=g, kk=kk):
                    wait_g(sidx, g, rbufs[kk], gsems[kk])
                    accumulate(dloc, rbufs[kk], g)

            return 0

        lax.fori_loop(0, (ng + NBUF - 1) // NBUF, inner, 0)

    def pass_body(p, _):
        blk = p * NW + wid
        base = blk * BR
        pltpu.sync_copy(zacc_hbm, acc)
        start_seg(0, 0, sem_s0)
        start_seg(1, 1, sem_s1)
        wait_seg(0, 0, sem_s0)
        ng_first = scan_into(0, 0, base)

        def seg_body(i, ngA):
            sA = i * 2
            fire_first(0, ngA)
            wait_seg(sA + 1, 1, sem_s1)
            ngB = scan_into(1, 1, base)

            @pl.when(sA + 2 < NSEG)
            def _():
                start_seg(sA + 2, 0, sem_s0)

            drain(0, ngA)
            fire_first(1, ngB)

            @pl.when(sA + 2 < NSEG)
            def _():
                wait_seg(sA + 2, 0, sem_s0)

            ngA2 = scan_into(0, 0, base)

            @pl.when(sA + 3 < NSEG)
            def _():
                start_seg(sA + 3, 1, sem_s1)

            drain(1, ngB)
            return ngA2

        lax.fori_loop(0, NSEG // 2, seg_body, ng_first)
        pltpu.sync_copy(acc.at[pl.ds(0, BR * D)],
                        out_hbm.at[pl.ds(blk * BR * D, BR * D)])
        return 0

    lax.fori_loop(0, NPASS, pass_body, 0)


def _tc1_body(dg, x, w, dinv_o, y_o):
    deg = dg[...][:, :1] + 1.0
    dinv = lax.rsqrt(deg)
    dinv_o[...] = jnp.broadcast_to(dinv, (ROWBLK, 128))
    y_o[...] = jnp.dot(x[...], w[...], preferred_element_type=jnp.float32) * dinv


def _tc1(dg, x, w):
    nb = N_NODES // ROWBLK
    return pl.pallas_call(
        _tc1_body,
        grid=(nb,),
        in_specs=[
            pl.BlockSpec((ROWBLK, 16), lambda i: (i, 0)),
            pl.BlockSpec((ROWBLK, D), lambda i: (i, 0)),
            pl.BlockSpec((D, D), lambda i: (0, 0)),
        ],
        out_specs=[
            pl.BlockSpec((ROWBLK, 128), lambda i: (i, 0)),
            pl.BlockSpec((ROWBLK, D), lambda i: (i, 0)),
        ],
        out_shape=[
            jax.ShapeDtypeStruct((N_NODES, 128), jnp.float32),
            jax.ShapeDtypeStruct((N_NODES, D), jnp.float32),
        ],
    )(dg, x, w)


def _tc2_body(sagg, y, dinv_r, b, w, ynext_o):
    dinv = dinv_r[...][:, :1]
    h = jnp.maximum(dinv * (sagg[...] + y[...]) + b[...], 0.0)
    ynext_o[...] = jnp.dot(h, w[...], preferred_element_type=jnp.float32) * dinv


def _tc2(sagg, y, dinv, b, w):
    nb = N_NODES // ROWBLK
    return pl.pallas_call(
        _tc2_body,
        grid=(nb,),
        in_specs=[
            pl.BlockSpec((ROWBLK, D), lambda i: (i, 0)),
            pl.BlockSpec((ROWBLK, D), lambda i: (i, 0)),
            pl.BlockSpec((ROWBLK, 128), lambda i: (i, 0)),
            pl.BlockSpec((1, D), lambda i: (0, 0)),
            pl.BlockSpec((D, D), lambda i: (0, 0)),
        ],
        out_specs=pl.BlockSpec((ROWBLK, D), lambda i: (i, 0)),
        out_shape=jax.ShapeDtypeStruct((N_NODES, D), jnp.float32),
    )(sagg, y, dinv, b, w)


def _tc3_body(sagg, y, dinv_r, b, batch, wc, bc, out_o, pool, cnt):
    i = pl.program_id(0)

    @pl.when(i == 0)
    def _():
        pool[...] = jnp.zeros_like(pool)
        cnt[...] = jnp.zeros_like(cnt)

    dinv = dinv_r[...][:, :1]
    h = jnp.maximum(dinv * (sagg[...] + y[...]) + b[...], 0.0)
    bt = batch[...].reshape(1, ROWBLK)
    gids = lax.broadcasted_iota(jnp.int32, (N_GRAPHS, ROWBLK), 0)
    mask = (bt == gids).astype(jnp.float32)
    pool[...] += jnp.dot(mask, h, preferred_element_type=jnp.float32)
    cnt[...] += jnp.dot(mask, jnp.ones((ROWBLK, 128), jnp.float32),
                        preferred_element_type=jnp.float32)

    @pl.when(i == pl.num_programs(0) - 1)
    def _():
        pooled = pool[...] / jnp.maximum(cnt[...][:, :1], 1.0)
        p = jnp.maximum(pooled, 0.0)
        out_o[...] = jnp.dot(p, wc[...], preferred_element_type=jnp.float32) + bc[...]


def _tc3(sagg, y, dinv, b, batch3, wcp, bcp):
    nb = N_NODES // ROWBLK
    return pl.pallas_call(
        _tc3_body,
        grid=(nb,),
        in_specs=[
            pl.BlockSpec((ROWBLK, D), lambda i: (i, 0)),
            pl.BlockSpec((ROWBLK, D), lambda i: (i, 0)),
            pl.BlockSpec((ROWBLK, 128), lambda i: (i, 0)),
            pl.BlockSpec((1, D), lambda i: (0, 0)),
            pl.BlockSpec((1, 1, ROWBLK), lambda i: (i, 0, 0)),
            pl.BlockSpec((D, 128), lambda i: (0, 0)),
            pl.BlockSpec((1, 128), lambda i: (0, 0)),
        ],
        out_specs=pl.BlockSpec((N_GRAPHS, 128), lambda i: (0, 0)),
        out_shape=jax.ShapeDtypeStruct((N_GRAPHS, 128), jnp.float32),
        scratch_shapes=[
            pltpu.VMEM((N_GRAPHS, D), jnp.float32),
            pltpu.VMEM((N_GRAPHS, 128), jnp.float32),
        ],
    )(sagg, y, dinv, b, batch3, wcp, bcp)


def kernel(x_graph, edge_index_graph, batch, edge_attr, pos,
           W1, b1, W2, b2, W3, b3, Wc, bc):
    del edge_attr, pos
    src = edge_index_graph[0]
    dst = edge_index_graph[1]

    zacc = jnp.zeros((ACCW,), jnp.float32)
    batch3 = batch.reshape(N_NODES // ROWBLK, 1, ROWBLK)
    b1r = b1.reshape(1, D)
    b2r = b2.reshape(1, D)
    b3r = b3.reshape(1, D)
    wcp = jnp.pad(Wc, ((0, 0), (0, 128 - Wc.shape[1])))
    bcp = jnp.pad(bc, (0, 128 - bc.shape[0])).reshape(1, 128)

    deg16 = _sc_degree(dst)
    dinv, y1 = _tc1(deg16, x_graph, W1)
    s1 = _sc_propagate(y1, src, dst, zacc).reshape(N_NODES, D)
    y2 = _tc2(s1, y1, dinv, b1r, W2)
    s2 = _sc_propagate(y2, src, dst, zacc).reshape(N_NODES, D)
    y3 = _tc2(s2, y2, dinv, b2r, W3)
    s3 = _sc_propagate(y3, src, dst, zacc).reshape(N_NODES, D)
    logits128 = _tc3(s3, y3, dinv, b3r, batch3, wcp, bcp)
    return logits128[:, :Wc.shape[1]]


# speedup vs baseline: 1.1673x; 1.1673x over previous
import functools

import jax
import jax.numpy as jnp
from jax import lax
from jax.experimental import pallas as pl
from jax.experimental.pallas import tpu as pltpu
from jax.experimental.pallas import tpu_sc as plsc

N_NODES = 16384
N_EDGES = 262144
D = 256
N_GRAPHS = 64
N_SC = 2
N_TILES = 16
NW = N_SC * N_TILES
SEG = 4096
NSEG = N_EDGES // SEG
BR = 256
NPASS = N_NODES // (BR * NW)
GR = 16
NBUF = 4
ACCW = (BR + 1) * D
DR = 512
ROWBLK = 1024

_mesh = plsc.VectorSubcoreMesh(core_axis_name="c", subcore_axis_name="s")
_sc_params = pltpu.CompilerParams(needs_layout_passes=False)


@functools.partial(
    pl.kernel,
    out_type=jax.ShapeDtypeStruct((N_NODES, 16), jnp.float32),
    mesh=_mesh,
    scratch_types=[
        pltpu.VMEM((2, SEG), jnp.int32),
        pltpu.VMEM((16 * DR + 16,), jnp.float32),
        pltpu.VMEM((DR, 16), jnp.float32),
        pltpu.SemaphoreType.DMA,
        pltpu.SemaphoreType.DMA,
    ],
    compiler_params=_sc_params,
)
def _sc_degree(dst_hbm, deg_hbm, dstb, hist, degrep, sem0, sem1):
    c = lax.axis_index("c")
    s = lax.axis_index("s")
    wid = c * N_TILES + s
    base = wid * DR

    def zero_hist(i, _):
        hist[pl.ds(i * 16, 16)] = jnp.zeros((16,), jnp.float32)
        return 0

    lax.fori_loop(0, (16 * DR + 16) // 16, zero_hist, 0)

    def start_seg(seg, k, sem):
        pltpu.async_copy(dst_hbm.at[pl.ds(seg * SEG, SEG)], dstb.at[k], sem)

    def wait_seg(seg, k, sem):
        pltpu.make_async_copy(dst_hbm.at[pl.ds(seg * SEG, SEG)],
                              dstb.at[k], sem).wait()

    ones = jnp.ones((16,), jnp.float32)
    lanes = lax.iota(jnp.int32, 16)

    def scan(k):
        def body(v, _):
            dv = dstb[k, pl.ds(v * 16, 16)]
            m = (dv >= base) & (dv < base + DR)
            idx = jnp.where(m, lanes * DR + (dv - base), 16 * DR + lanes)
            plsc.addupdate_scatter(hist, [idx], ones)
            return 0

        lax.fori_loop(0, SEG // 16, body, 0)

    start_seg(0, 0, sem0)

    def seg_body(i, _):
        g0 = i * 2
        start_seg(g0 + 1, 1, sem1)
        wait_seg(g0, 0, sem0)
        scan(0)

        @pl.when(g0 + 2 < NSEG)
        def _():
            start_seg(g0 + 2, 0, sem0)

        wait_seg(g0 + 1, 1, sem1)
        scan(1)
        return 0

    lax.fori_loop(0, NSEG // 2, seg_body, 0)

    for j in range(DR // 16):
        acc = hist[pl.ds(j * 16, 16)]
        for l in range(1, 16):
            acc = acc + hist[pl.ds(l * DR + j * 16, 16)]
        for l in range(16):
            degrep[j * 16 + l, pl.ds(0, 16)] = jnp.full((16,), acc[l],
                                                        jnp.float32)
    pltpu.sync_copy(degrep, deg_hbm.at[pl.ds(base, DR)])


@functools.partial(
    pl.kernel,
    out_type=jax.ShapeDtypeStruct((N_NODES * D,), jnp.float32),
    mesh=_mesh,
    scratch_types=[
        pltpu.VMEM((2, SEG), jnp.int32),
        pltpu.VMEM((2, SEG), jnp.int32),
        pltpu.VMEM((SEG + 128,), jnp.int32),
        pltpu.VMEM((SEG + 128,), jnp.int32),
        pltpu.VMEM((SEG + 128,), jnp.int32),
        pltpu.VMEM((SEG + 128,), jnp.int32),
    ] + [pltpu.VMEM((GR, D), jnp.float32)] * NBUF + [
        pltpu.VMEM((ACCW,), jnp.float32),
    ] + [pltpu.SemaphoreType.DMA] * (2 + NBUF),
    compiler_params=_sc_params,
)
def _sc_propagate(y_hbm, src_hbm, dst_hbm, zacc_hbm, out_hbm,
                  srcb, dstb, sidx0, dloc0, sidx1, dloc1, *rest):
    banks = ((sidx0, dloc0), (sidx1, dloc1))
    rbufs = rest[:NBUF]
    acc = rest[NBUF]
    sem_s0, sem_s1 = rest[NBUF + 1], rest[NBUF + 2]
    gsems = rest[NBUF + 3:]
    c = lax.axis_index("c")
    s = lax.axis_index("s")
    wid = c * N_TILES + s
    lanes = lax.iota(jnp.int32, 16)

    def start_seg(seg, k, sem):
        pltpu.async_copy(src_hbm.at[pl.ds(seg * SEG, SEG)], srcb.at[k], sem)
        pltpu.async_copy(dst_hbm.at[pl.ds(seg * SEG, SEG)], dstb.at[k], sem)

    def wait_seg(seg, k, sem):
        pltpu.make_async_copy(src_hbm.at[pl.ds(seg * SEG, SEG)],
                              srcb.at[k], sem).wait()
        pltpu.make_async_copy(dst_hbm.at[pl.ds(seg * SEG, SEG)],
                              dstb.at[k], sem).wait()

    def start_g(sidx, g, rbuf, sem):
        pltpu.async_copy(y_hbm.at[sidx.at[pl.ds(g * GR, GR)]], rbuf, sem)

    def wait_g(sidx, g, rbuf, sem):
        pltpu.make_async_copy(y_hbm.at[sidx.at[pl.ds(g * GR, GR)]],
                              rbuf, sem).wait()

    def accumulate(dloc, rbuf, g):
        def rbody(r, _):
            dl16 = dloc[pl.ds(g * GR + r, 16)]
            off = dl16[0] * D
            for j in range(D // 16):
                vec = rbuf[r, pl.ds(j * 16, 16)]
                plsc.addupdate(acc.at[pl.ds(off + j * 16, 16)], vec)
            return 0

        lax.fori_loop(0, GR, rbody, 0)

    def scan_into(b, k, base):
        sidx, dloc = banks[b]

        def scan_body(v, cnt):
            dvs, svs, mis, incls = [], [], [], []
            for u in range(4):
                dv = dstb[k, pl.ds((v * 4 + u) * 16, 16)]
                sv = srcb[k, pl.ds((v * 4 + u) * 16, 16)]
                m = (dv >= base) & (dv < base + BR)
                mi = m.astype(jnp.int32)
                incls.append(plsc.cumsum(mi))
                dvs.append(dv)
                svs.append(sv)
                mis.append(mi)
            for u in range(4):
                m = mis[u].astype(jnp.bool_)
                pos = jnp.where(m, cnt + incls[u] - mis[u], SEG + lanes)
                plsc.store_scatter(sidx, [pos], svs[u])
                plsc.store_scatter(dloc, [pos], dvs[u] - base)
                cnt = cnt + incls[u][15]
            return cnt

        cnt = lax.fori_loop(0, SEG // 64, scan_body, 0)
        for pvi in range(GR // 16):
            sidx[pl.ds(cnt + pvi * 16, 16)] = jnp.zeros((16,), jnp.int32)
            dloc[pl.ds(cnt + pvi * 16, 16)] = jnp.full((16,), BR, jnp.int32)
        return (cnt + GR - 1) // GR

    def fire_first(b, ng):
        sidx, _ = banks[b]
        for kk in range(NBUF - 1):
            @pl.when(kk < ng)
            def _(kk=kk):
                start_g(sidx, kk, rbufs[kk], gsems[kk])

    def drain(b, ng):
        sidx, dloc = banks[b]

        def inner(jj, _):
            for kk in range(NBUF):
                g = jj * NBUF + kk

                @pl.when(g + NBUF - 1 < ng)
                def _(g=g, kk=kk):
                    start_g(sidx, g + NBUF - 1, rbufs[(kk + NBUF - 1) % NBUF],
                            gsems[(kk + NBUF - 1) % NBUF])

                @pl.when(g < ng)
                def _(g=g, kk=kk):
                    wait_g(sidx, g, rbufs[kk], gsems[kk])
                    accumulate(dloc, rbufs[kk], g)

            return 0

        lax.fori_loop(0, (ng + NBUF - 1) // NBUF, inner, 0)

    def pass_body(p, _):
        blk = p * NW + wid
        base = blk * BR
        pltpu.sync_copy(zacc_hbm, acc)
        start_seg(0, 0, sem_s0)
        start_seg(1, 1, sem_s1)
        wait_seg(0, 0, sem_s0)
        ng_first = scan_into(0, 0, base)

        def seg_body(i, ngA):
            sA = i * 2
            fire_first(0, ngA)
            wait_seg(sA + 1, 1, sem_s1)
            ngB = scan_into(1, 1, base)

            @pl.when(sA + 2 < NSEG)
            def _():
                start_seg(sA + 2, 0, sem_s0)

            drain(0, ngA)
            fire_first(1, ngB)

            @pl.when(sA + 2 < NSEG)
            def _():
                wait_seg(sA + 2, 0, sem_s0)

            ngA2 = scan_into(0, 0, base)

            @pl.when(sA + 3 < NSEG)
            def _():
                start_seg(sA + 3, 1, sem_s1)

            drain(1, ngB)
            return ngA2

        lax.fori_loop(0, NSEG // 2, seg_body, ng_first)
        pltpu.sync_copy(acc.at[pl.ds(0, BR * D)],
                        out_hbm.at[pl.ds(blk * BR * D, BR * D)])
        return 0

    lax.fori_loop(0, NPASS, pass_body, 0)


def _tc1_body(dg, x, w, dinv_o, y_o):
    deg = dg[...][:, :1] + 1.0
    dinv = lax.rsqrt(deg)
    dinv_o[...] = jnp.broadcast_to(dinv, (ROWBLK, 128))
    y_o[...] = jnp.dot(x[...], w[...], preferred_element_type=jnp.float32) * dinv


def _tc1(dg, x, w):
    nb = N_NODES // ROWBLK
    return pl.pallas_call(
        _tc1_body,
        grid=(nb,),
        in_specs=[
            pl.BlockSpec((ROWBLK, 16), lambda i: (i, 0)),
            pl.BlockSpec((ROWBLK, D), lambda i: (i, 0)),
            pl.BlockSpec((D, D), lambda i: (0, 0)),
        ],
        out_specs=[
            pl.BlockSpec((ROWBLK, 128), lambda i: (i, 0)),
            pl.BlockSpec((ROWBLK, D), lambda i: (i, 0)),
        ],
        out_shape=[
            jax.ShapeDtypeStruct((N_NODES, 128), jnp.float32),
            jax.ShapeDtypeStruct((N_NODES, D), jnp.float32),
        ],
    )(dg, x, w)


def _tc2_body(sagg, y, dinv_r, b, w, ynext_o):
    dinv = dinv_r[...][:, :1]
    h = jnp.maximum(dinv * (sagg[...] + y[...]) + b[...], 0.0)
    ynext_o[...] = jnp.dot(h, w[...], preferred_element_type=jnp.float32) * dinv


def _tc2(sagg, y, dinv, b, w):
    nb = N_NODES // ROWBLK
    return pl.pallas_call(
        _tc2_body,
        grid=(nb,),
        in_specs=[
            pl.BlockSpec((ROWBLK, D), lambda i: (i, 0)),
            pl.BlockSpec((ROWBLK, D), lambda i: (i, 0)),
            pl.BlockSpec((ROWBLK, 128), lambda i: (i, 0)),
            pl.BlockSpec((1, D), lambda i: (0, 0)),
            pl.BlockSpec((D, D), lambda i: (0, 0)),
        ],
        out_specs=pl.BlockSpec((ROWBLK, D), lambda i: (i, 0)),
        out_shape=jax.ShapeDtypeStruct((N_NODES, D), jnp.float32),
    )(sagg, y, dinv, b, w)


def _tc3_body(sagg, y, dinv_r, b, batch, wc, bc, out_o, pool, cnt):
    i = pl.program_id(0)

    @pl.when(i == 0)
    def _():
        pool[...] = jnp.zeros_like(pool)
        cnt[...] = jnp.zeros_like(cnt)

    dinv = dinv_r[...][:, :1]
    h = jnp.maximum(dinv * (sagg[...] + y[...]) + b[...], 0.0)
    bt = batch[...].reshape(1, ROWBLK)
    gids = lax.broadcasted_iota(jnp.int32, (N_GRAPHS, ROWBLK), 0)
    mask = (bt == gids).astype(jnp.float32)
    pool[...] += jnp.dot(mask, h, preferred_element_type=jnp.float32)
    cnt[...] += jnp.dot(mask, jnp.ones((ROWBLK, 128), jnp.float32),
                        preferred_element_type=jnp.float32)

    @pl.when(i == pl.num_programs(0) - 1)
    def _():
        pooled = pool[...] / jnp.maximum(cnt[...][:, :1], 1.0)
        p = jnp.maximum(pooled, 0.0)
        out_o[...] = jnp.dot(p, wc[...], preferred_element_type=jnp.float32) + bc[...]


def _tc3(sagg, y, dinv, b, batch3, wcp, bcp):
    nb = N_NODES // ROWBLK
    return pl.pallas_call(
        _tc3_body,
        grid=(nb,),
        in_specs=[
            pl.BlockSpec((ROWBLK, D), lambda i: (i, 0)),
            pl.BlockSpec((ROWBLK, D), lambda i: (i, 0)),
            pl.BlockSpec((ROWBLK, 128), lambda i: (i, 0)),
            pl.BlockSpec((1, D), lambda i: (0, 0)),
            pl.BlockSpec((1, 1, ROWBLK), lambda i: (i, 0, 0)),
            pl.BlockSpec((D, 128), lambda i: (0, 0)),
            pl.BlockSpec((1, 128), lambda i: (0, 0)),
        ],
        out_specs=pl.BlockSpec((N_GRAPHS, 128), lambda i: (0, 0)),
        out_shape=jax.ShapeDtypeStruct((N_GRAPHS, 128), jnp.float32),
        scratch_shapes=[
            pltpu.VMEM((N_GRAPHS, D), jnp.float32),
            pltpu.VMEM((N_GRAPHS, 128), jnp.float32),
        ],
    )(sagg, y, dinv, b, batch3, wcp, bcp)


def kernel(x_graph, edge_index_graph, batch, edge_attr, pos,
           W1, b1, W2, b2, W3, b3, Wc, bc):
    del edge_attr, pos
    src = edge_index_graph[0]
    dst = edge_index_graph[1]

    zacc = jnp.zeros((ACCW,), jnp.float32)
    batch3 = batch.reshape(N_NODES // ROWBLK, 1, ROWBLK)
    b1r = b1.reshape(1, D)
    b2r = b2.reshape(1, D)
    b3r = b3.reshape(1, D)
    wcp = jnp.pad(Wc, ((0, 0), (0, 128 - Wc.shape[1])))
    bcp = jnp.pad(bc, (0, 128 - bc.shape[0])).reshape(1, 128)

    deg16 = _sc_degree(dst)
    dinv, y1 = _tc1(deg16, x_graph, W1)
    s1 = _sc_propagate(y1, src, dst, zacc).reshape(N_NODES, D)
    y2 = _tc2(s1, y1, dinv, b1r, W2)
    s2 = _sc_propagate(y2, src, dst, zacc).reshape(N_NODES, D)
    y3 = _tc2(s2, y2, dinv, b2r, W3)
    s3 = _sc_propagate(y3, src, dst, zacc).reshape(N_NODES, D)
    logits128 = _tc3(s3, y3, dinv, b3r, batch3, wcp, bcp)
    return logits128[:, :Wc.shape[1]]
